# baseline (speedup 1.0000x reference)
import functools

import jax
import jax.numpy as jnp
from jax import lax
from jax.experimental import pallas as pl
from jax.experimental.pallas import tpu as pltpu
from jax.experimental.pallas import tpu_sc as plsc

N = 10000
E = 320000
F = 16

TILES = 16
CH = 128
NCH = 160
BC = 4
NCHB = NCH // BC
DEPTH = 4
LA = DEPTH // 2
NCHP = NCH + LA * BC
EPT = NCH * CH
E_PAD = TILES * EPT
ACC_ROWS = 10112
ZROWS = ACC_ROWS // TILES
FROWS = 624
TAIL = N - TILES * FROWS


def _sc_body(round_outs, fa_pad, srcm, dstm, *rest):
    nouts = sum(1 for o in round_outs if o)
    outs = rest[:nouts]
    refs = rest[nouts:]
    src_v, dst_v = refs[0], refs[1]
    rows = refs[2:2 + DEPTH]
    zero_v, flush_v, bufa, bufb = refs[2 + DEPTH:6 + DEPTH]
    gsem = refs[6 + DEPTH:6 + 2 * DEPTH]
    ssem = refs[6 + 2 * DEPTH:6 + 3 * DEPTH]
    c = lax.axis_index("c")
    s = lax.axis_index("s")
    out_it = iter(outs)
    rounds = [(r + 1, next(out_it) if o else None)
              for r, o in enumerate(round_outs)]

    @pl.when(c == 0)
    def _():
        pltpu.sync_copy(srcm.at[s], src_v)
        pltpu.sync_copy(dstm.at[s], dst_v)

        def _zrow(i, carry):
            zero_v[i, :] = jnp.zeros((F,), jnp.float32)
            return carry
        lax.fori_loop(0, ZROWS, _zrow, 0)
        pltpu.sync_copy(fa_pad.at[pl.ds(s * ZROWS, ZROWS)], flush_v)
        pltpu.sync_copy(flush_v, bufa.at[pl.ds(s * ZROWS, ZROWS)])
        pltpu.sync_copy(zero_v, bufb.at[pl.ds(s * ZROWS, ZROWS)])
        plsc.subcore_barrier()

        nrounds = len(rounds)
        for rnd, zout in rounds:
            table = bufa if rnd % 2 == 1 else bufb
            accb = bufb if rnd % 2 == 1 else bufa

            def gst(j, b, table=table):
                pltpu.async_copy(table.at[src_v.at[pl.ds(j * BC * CH,
                                                         BC * CH)]],
                                 rows[b], gsem[b])

            def gwt(j, b, table=table):
                pltpu.make_async_copy(table.at[src_v.at[pl.ds(j * BC * CH,
                                                              BC * CH)]],
                                      rows[b], gsem[b]).wait()

            def sst(j, b, accb=accb):
                for u in range(BC):
                    pltpu.async_copy(rows[b].at[pl.ds(u * CH, CH)],
                                     accb.at[dst_v.at[j * BC + u]],
                                     ssem[b], add=True)

            def swt(j, b, accb=accb):
                for u in range(BC):
                    pltpu.make_async_copy(rows[b].at[pl.ds(u * CH, CH)],
                                          accb.at[dst_v.at[j * BC + u]],
                                          ssem[b]).wait()

            for j in range(LA):
                gst(j, j % DEPTH)
            for j in range(DEPTH):
                gwt(j, j % DEPTH)
                if j - LA >= 0:
                    swt(j - LA, (j - LA) % DEPTH)
                gst(j + LA, (j + LA) % DEPTH)
                sst(j, j % DEPTH)

            def _body(p, carry):
                j0 = DEPTH * p
                for u in range(DEPTH):
                    j = j0 + u
                    gwt(j, u)
                    swt(j - LA, (u - LA) % DEPTH)
                    gst(j + LA, (u + LA) % DEPTH)
                    sst(j, u)
                return carry
            lax.fori_loop(1, NCHB // DEPTH, _body, 0)
            for j in range(NCHB, NCHB + LA):
                gwt(j, j % DEPTH)
            for j in range(NCHB - LA, NCHB):
                swt(j, j % DEPTH)
            plsc.subcore_barrier()
            if zout is not None:
                pltpu.sync_copy(accb.at[pl.ds(s * FROWS, FROWS)],
                                flush_v.at[pl.ds(0, FROWS)])
                pltpu.sync_copy(flush_v.at[pl.ds(0, FROWS)],
                                zout.at[pl.ds(s * FROWS, FROWS)])

                @pl.when(s == 0)
                def _(accb=accb, zout=zout):
                    pltpu.sync_copy(accb.at[pl.ds(TILES * FROWS, TAIL)],
                                    flush_v.at[pl.ds(0, TAIL)])
                    pltpu.sync_copy(flush_v.at[pl.ds(0, TAIL)],
                                    zout.at[pl.ds(TILES * FROWS, TAIL)])
            if rnd < nrounds:
                pltpu.sync_copy(zero_v, table.at[pl.ds(s * ZROWS, ZROWS)])
                plsc.subcore_barrier()


_zshape = jax.ShapeDtypeStruct((N, F), jnp.float32)


@functools.lru_cache(maxsize=None)
def _sc_segsum(round_outs=(True, True, False, True)):
    return pl.kernel(
        functools.partial(_sc_body, round_outs),
        out_type=tuple(_zshape for o in round_outs if o),
        mesh=plsc.VectorSubcoreMesh(core_axis_name="c", subcore_axis_name="s"),
        compiler_params=pltpu.CompilerParams(use_tc_tiling_on_sc=False),
        scratch_types=[
            pltpu.VMEM((NCHP * CH,), jnp.int32),
            pltpu.VMEM((NCHP, CH), jnp.int32),
        ] + [pltpu.VMEM((BC * CH, F), jnp.float32)] * DEPTH + [
            pltpu.VMEM((ZROWS, F), jnp.float32),
            pltpu.VMEM((ZROWS, F), jnp.float32),
            pltpu.VMEM_SHARED((ACC_ROWS, F), jnp.float32),
            pltpu.VMEM_SHARED((ACC_ROWS, F), jnp.float32),
        ] + [pltpu.SemaphoreType.DMA] * (2 * DEPTH),
    )


BLK = 400
NBLK = N // BLK


def _tc_sweep_body(pm, fa, dg, fb, Wp, Wd, Wf, bsum, out):
    fbw = jnp.dot(fb[...], Wf[...], preferred_element_type=jnp.float32)
    rows = jnp.dot(pm[...], fbw, preferred_element_type=jnp.float32)
    a = fa[...]
    rows += jnp.dot(a, Wp[...], preferred_element_type=jnp.float32)
    rows += jnp.dot(dg[...] * a, Wd[...], preferred_element_type=jnp.float32)
    out[...] = rows + bsum[...]


_tc_sweep = pl.pallas_call(
    _tc_sweep_body,
    grid=(NBLK,),
    in_specs=[
        pl.BlockSpec((BLK, N), lambda i: (i, 0)),
        pl.BlockSpec((BLK, F), lambda i: (i, 0)),
        pl.BlockSpec((BLK, 1), lambda i: (i, 0)),
        pl.BlockSpec((N, F), lambda i: (0, 0)),
        pl.BlockSpec((F, F), lambda i: (0, 0)),
        pl.BlockSpec((F, F), lambda i: (0, 0)),
        pl.BlockSpec((F, F), lambda i: (0, 0)),
        pl.BlockSpec((1, F), lambda i: (0, 0)),
    ],
    out_specs=pl.BlockSpec((BLK, F), lambda i: (i, 0)),
    out_shape=jax.ShapeDtypeStruct((N, F), jnp.float32),
    compiler_params=pltpu.CompilerParams(
        dimension_semantics=("arbitrary",),
        vmem_limit_bytes=100 * 1024 * 1024,
    ),
)


def _tc_combine_body(sw, z1, z2, z4, Wa, gam, bet, out):
    rows = sw[...]
    rows += jnp.dot(z1[...], Wa[0], preferred_element_type=jnp.float32)
    rows += jnp.dot(z2[...], Wa[1], preferred_element_type=jnp.float32)
    rows += jnp.dot(z4[...], Wa[2], preferred_element_type=jnp.float32)
    col = lax.broadcasted_iota(jnp.int32, rows.shape, 1)
    rows = jnp.where(col >= F // 2, jnp.maximum(rows, 0.0), rows)
    mean = jnp.mean(rows, axis=0, keepdims=True)
    xc = rows - mean
    var = jnp.mean(xc * xc, axis=0, keepdims=True)
    out[...] = gam[...] * xc * lax.rsqrt(var + 1e-5) + bet[...]


_tc_combine = pl.pallas_call(
    _tc_combine_body,
    in_specs=[pl.BlockSpec(memory_space=pltpu.VMEM)] * 7,
    out_specs=pl.BlockSpec(memory_space=pltpu.VMEM),
    out_shape=jax.ShapeDtypeStruct((N, F), jnp.float32),
)


def kernel(feat_a, feat_b, deg, pm_pd, edge_index,
           W_prev, b_prev, W_deg, b_deg, W_agg, b_agg,
           W_fuse, b_fuse, gamma, beta):
    src = edge_index[0]
    dst = edge_index[1]
    pad = E_PAD - E
    src_p = jnp.concatenate([src, jnp.zeros((pad,), jnp.int32)])
    dst_p = jnp.concatenate([dst, jnp.full((pad,), N, jnp.int32)])
    srcm = jnp.concatenate(
        [src_p.reshape(TILES, NCH * CH),
         jnp.zeros((TILES, (NCHP - NCH) * CH), jnp.int32)], axis=1)
    dstm = jnp.concatenate(
        [dst_p.reshape(TILES, NCH, CH),
         jnp.full((TILES, NCHP - NCH, CH), N, jnp.int32)], axis=1)

    fa_pad = jnp.concatenate(
        [feat_a, jnp.zeros((ACC_ROWS - N, F), jnp.float32)])
    z1, z2, z4 = _sc_segsum((True, True, False, True))(fa_pad, srcm, dstm)

    bsum = (b_prev + b_deg + b_agg.sum(axis=0) + b_fuse).reshape(1, F)
    sweep = _tc_sweep(pm_pd, feat_a, deg, feat_b, W_prev, W_deg, W_fuse, bsum)
    return _tc_combine(sweep, z1, z2, z4, W_agg,
                       gamma.reshape(1, F), beta.reshape(1, F))


# speedup vs baseline: 1.1458x; 1.1458x over previous
import functools

import jax
import jax.numpy as jnp
from jax import lax
from jax.experimental import pallas as pl
from jax.experimental.pallas import tpu as pltpu
from jax.experimental.pallas import tpu_sc as plsc

N = 10000
E = 320000
F = 16

TILES = 16
CH = 128
NCH = 160
DEPTH = 4
NCHP = NCH + DEPTH // 2
EPT = NCH * CH
E_PAD = TILES * EPT
ACC_ROWS = 10112
ZROWS = ACC_ROWS // TILES
FROWS = 624
TAIL = N - TILES * FROWS


def _sc_body(fa_pad, srcm, dstm, z1, z2, z4, *refs):
    src_v, dst_v = refs[0], refs[1]
    rows = refs[2:2 + DEPTH]
    zero_v, flush_v, bufa, bufb = refs[2 + DEPTH:6 + DEPTH]
    gsem = refs[6 + DEPTH:6 + 2 * DEPTH]
    ssem = refs[6 + 2 * DEPTH:6 + 3 * DEPTH]
    c = lax.axis_index("c")
    s = lax.axis_index("s")

    @pl.when(c == 0)
    def _():
        pltpu.sync_copy(srcm.at[s], src_v)
        pltpu.sync_copy(dstm.at[s], dst_v)

        def _zrow(i, carry):
            zero_v[i, :] = jnp.zeros((F,), jnp.float32)
            return carry
        lax.fori_loop(0, ZROWS, _zrow, 0)
        pltpu.sync_copy(fa_pad.at[pl.ds(s * ZROWS, ZROWS)], flush_v)
        pltpu.sync_copy(flush_v, bufa.at[pl.ds(s * ZROWS, ZROWS)])
        pltpu.sync_copy(zero_v, bufb.at[pl.ds(s * ZROWS, ZROWS)])
        plsc.subcore_barrier()

        for rnd, zout in ((1, z1), (2, z2), (3, None), (4, z4)):
            table = bufa if rnd % 2 == 1 else bufb
            accb = bufb if rnd % 2 == 1 else bufa

            def gst(j, b, table=table):
                pltpu.async_copy(table.at[src_v.at[j]], rows[b], gsem[b])

            def gwt(j, b, table=table):
                pltpu.make_async_copy(
                    table.at[src_v.at[j]], rows[b], gsem[b]).wait()

            def sst(j, b, accb=accb):
                pltpu.async_copy(rows[b], accb.at[dst_v.at[j]], ssem[b],
                                 add=True)

            def swt(j, b, accb=accb):
                pltpu.make_async_copy(
                    rows[b], accb.at[dst_v.at[j]], ssem[b]).wait()

            LA = DEPTH // 2
            for j in range(LA):
                gst(j, j % DEPTH)
            for j in range(DEPTH):
                gwt(j, j % DEPTH)
                if j - LA >= 0:
                    swt(j - LA, (j - LA) % DEPTH)
                gst(j + LA, (j + LA) % DEPTH)
                sst(j, j % DEPTH)

            def _body(p, carry):
                j0 = DEPTH * p
                for u in range(DEPTH):
                    j = j0 + u
                    gwt(j, u)
                    swt(j - LA, (u - LA) % DEPTH)
                    gst(j + LA, (u + LA) % DEPTH)
                    sst(j, u)
                return carry
            lax.fori_loop(1, NCH // DEPTH, _body, 0)
            for j in range(NCH, NCH + LA):
                gwt(j, j % DEPTH)
            for j in range(NCH - LA, NCH):
                swt(j, j % DEPTH)
            plsc.subcore_barrier()
            if zout is not None:
                pltpu.sync_copy(accb.at[pl.ds(s * FROWS, FROWS)],
                                flush_v.at[pl.ds(0, FROWS)])
                pltpu.sync_copy(flush_v.at[pl.ds(0, FROWS)],
                                zout.at[pl.ds(s * FROWS, FROWS)])

                @pl.when(s == 0)
                def _(accb=accb, zout=zout):
                    pltpu.sync_copy(accb.at[pl.ds(TILES * FROWS, TAIL)],
                                    flush_v.at[pl.ds(0, TAIL)])
                    pltpu.sync_copy(flush_v.at[pl.ds(0, TAIL)],
                                    zout.at[pl.ds(TILES * FROWS, TAIL)])
            if rnd < 4:
                pltpu.sync_copy(zero_v, table.at[pl.ds(s * ZROWS, ZROWS)])
                plsc.subcore_barrier()


_zshape = jax.ShapeDtypeStruct((N, F), jnp.float32)


@functools.lru_cache(maxsize=None)
def _sc_segsum():
    return pl.kernel(
        _sc_body,
        out_type=(_zshape, _zshape, _zshape),
        mesh=plsc.VectorSubcoreMesh(core_axis_name="c", subcore_axis_name="s"),
        compiler_params=pltpu.CompilerParams(use_tc_tiling_on_sc=False),
        scratch_types=[
            pltpu.VMEM((NCHP, CH), jnp.int32),
            pltpu.VMEM((NCHP, CH), jnp.int32),
        ] + [pltpu.VMEM((CH, F), jnp.float32)] * DEPTH + [
            pltpu.VMEM((ZROWS, F), jnp.float32),
            pltpu.VMEM((ZROWS, F), jnp.float32),
            pltpu.VMEM_SHARED((ACC_ROWS, F), jnp.float32),
            pltpu.VMEM_SHARED((ACC_ROWS, F), jnp.float32),
        ] + [pltpu.SemaphoreType.DMA] * (2 * DEPTH),
    )


BLK = 400
NBLK = N // BLK


def _tc_sweep_body(pm, fa, dg, fb, Wp, Wd, Wf, bsum, out):
    fbw = jnp.dot(fb[...], Wf[...], preferred_element_type=jnp.float32)
    rows = jnp.dot(pm[...], fbw, preferred_element_type=jnp.float32)
    a = fa[...]
    rows += jnp.dot(a, Wp[...], preferred_element_type=jnp.float32)
    rows += jnp.dot(dg[...] * a, Wd[...], preferred_element_type=jnp.float32)
    out[...] = rows + bsum[...]


_tc_sweep = pl.pallas_call(
    _tc_sweep_body,
    grid=(NBLK,),
    in_specs=[
        pl.BlockSpec((BLK, N), lambda i: (i, 0)),
        pl.BlockSpec((BLK, F), lambda i: (i, 0)),
        pl.BlockSpec((BLK, 1), lambda i: (i, 0)),
        pl.BlockSpec((N, F), lambda i: (0, 0)),
        pl.BlockSpec((F, F), lambda i: (0, 0)),
        pl.BlockSpec((F, F), lambda i: (0, 0)),
        pl.BlockSpec((F, F), lambda i: (0, 0)),
        pl.BlockSpec((1, F), lambda i: (0, 0)),
    ],
    out_specs=pl.BlockSpec((BLK, F), lambda i: (i, 0)),
    out_shape=jax.ShapeDtypeStruct((N, F), jnp.float32),
    compiler_params=pltpu.CompilerParams(
        dimension_semantics=("arbitrary",),
        vmem_limit_bytes=100 * 1024 * 1024,
    ),
)


def _tc_combine_body(sw, z1, z2, z4, Wa, gam, bet, out):
    rows = sw[...]
    rows += jnp.dot(z1[...], Wa[0], preferred_element_type=jnp.float32)
    rows += jnp.dot(z2[...], Wa[1], preferred_element_type=jnp.float32)
    rows += jnp.dot(z4[...], Wa[2], preferred_element_type=jnp.float32)
    col = lax.broadcasted_iota(jnp.int32, rows.shape, 1)
    rows = jnp.where(col >= F // 2, jnp.maximum(rows, 0.0), rows)
    mean = jnp.mean(rows, axis=0, keepdims=True)
    xc = rows - mean
    var = jnp.mean(xc * xc, axis=0, keepdims=True)
    out[...] = gam[...] * xc * lax.rsqrt(var + 1e-5) + bet[...]


_tc_combine = pl.pallas_call(
    _tc_combine_body,
    in_specs=[pl.BlockSpec(memory_space=pltpu.VMEM)] * 7,
    out_specs=pl.BlockSpec(memory_space=pltpu.VMEM),
    out_shape=jax.ShapeDtypeStruct((N, F), jnp.float32),
)


def kernel(feat_a, feat_b, deg, pm_pd, edge_index,
           W_prev, b_prev, W_deg, b_deg, W_agg, b_agg,
           W_fuse, b_fuse, gamma, beta):
    src = edge_index[0]
    dst = edge_index[1]
    pad = E_PAD - E
    src_p = jnp.concatenate([src, jnp.zeros((pad,), jnp.int32)])
    dst_p = jnp.concatenate([dst, jnp.full((pad,), N, jnp.int32)])
    srcm = jnp.concatenate(
        [src_p.reshape(TILES, NCH, CH),
         jnp.zeros((TILES, NCHP - NCH, CH), jnp.int32)], axis=1)
    dstm = jnp.concatenate(
        [dst_p.reshape(TILES, NCH, CH),
         jnp.full((TILES, NCHP - NCH, CH), N, jnp.int32)], axis=1)

    fa_pad = jnp.concatenate(
        [feat_a, jnp.zeros((ACC_ROWS - N, F), jnp.float32)])
    z1, z2, z4 = _sc_segsum()(fa_pad, srcm, dstm)

    bsum = (b_prev + b_deg + b_agg.sum(axis=0) + b_fuse).reshape(1, F)
    sweep = _tc_sweep(pm_pd, feat_a, deg, feat_b, W_prev, W_deg, W_fuse, bsum)
    return _tc_combine(sweep, z1, z2, z4, W_agg,
                       gamma.reshape(1, F), beta.reshape(1, F))
